# baseline (speedup 1.0000x reference)
import functools

import jax
import jax.numpy as jnp
from jax import lax
from jax.experimental import pallas as pl
from jax.experimental.pallas import tpu as pltpu
from jax.experimental.pallas import tpu_sc as plsc

_NC = 2
_NS = 16
_L = 16
_NW = _NC * _NS

_MESH = dict(core_axis_name="c", subcore_axis_name="s")


def _degree_partials(cols4, n_pad):
    nw, nwin, wchunk, ch = cols4.shape
    zps = n_pad // _NS

    @functools.partial(
        pl.kernel,
        out_type=jax.ShapeDtypeStruct((_NC * n_pad,), jnp.float32),
        mesh=plsc.VectorSubcoreMesh(**_MESH),
        scratch_types=[
            pltpu.VMEM((wchunk, ch), jnp.int32),
            pltpu.VMEM((ch,), jnp.float32),
            pltpu.VMEM((zps,), jnp.float32),
            pltpu.VMEM_SHARED((n_pad,), jnp.float32),
            pltpu.SemaphoreType.DMA,
        ],
    )
    def deg_kernel(cols_hbm, out_hbm, cidx, ones_v, zeros_v, deg_sh, sem):
        cid = lax.axis_index("c")
        sid = lax.axis_index("s")
        wid = cid * _NS + sid

        @pl.loop(0, ch, step=_L)
        def _(i):
            ones_v[pl.ds(i, _L)] = jnp.ones((_L,), jnp.float32)

        @pl.loop(0, zps, step=_L)
        def _(i):
            zeros_v[pl.ds(i, _L)] = jnp.zeros((_L,), jnp.float32)

        pltpu.sync_copy(zeros_v, deg_sh.at[pl.ds(sid * zps, zps)])
        plsc.subcore_barrier()

        @pl.loop(0, nwin)
        def _(w):
            pltpu.sync_copy(cols_hbm.at[wid, w], cidx)

            @pl.loop(0, wchunk)
            def _(j):
                pltpu.sync_copy(ones_v, deg_sh.at[cidx.at[j]], add=True)

        plsc.subcore_barrier()
        pltpu.sync_copy(deg_sh.at[pl.ds(sid * zps, zps)], zeros_v)
        pltpu.sync_copy(zeros_v, out_hbm.at[pl.ds(cid * n_pad + sid * zps, zps)])

    return deg_kernel(cols4)


def _aggregate(y, rows4, cols4, n_acc):
    n, d = y.shape
    nw, nwin, wchunk, ch = rows4.shape
    npc = n_acc // _NS
    nring = 4
    rem = wchunk % nring
    main_hi = wchunk - nring - rem

    @functools.partial(
        pl.kernel,
        out_type=jax.ShapeDtypeStruct((_NC, n_acc, d), jnp.float32),
        mesh=plsc.VectorSubcoreMesh(**_MESH),
        scratch_types=[
            pltpu.VMEM((wchunk, ch), jnp.int32),
            pltpu.VMEM((wchunk, ch), jnp.int32),
            pltpu.VMEM((ch, d), jnp.float32),
            pltpu.VMEM((ch, d), jnp.float32),
            pltpu.VMEM((ch, d), jnp.float32),
            pltpu.VMEM((ch, d), jnp.float32),
            pltpu.VMEM_SHARED((n_acc, d), jnp.float32),
            pltpu.SemaphoreType.DMA,
            pltpu.SemaphoreType.DMA,
            pltpu.SemaphoreType.DMA,
            pltpu.SemaphoreType.DMA,
            pltpu.SemaphoreType.DMA,
        ],
    )
    def agg_kernel(y_hbm, rows_hbm, cols_hbm, out_hbm,
                   ridx, cidx, b0, b1, b2, b3, agg_sh,
                   semi, s0, s1, s2, s3):
        bufs = (b0, b1, b2, b3)
        sems = (s0, s1, s2, s3)
        cid = lax.axis_index("c")
        sid = lax.axis_index("s")
        wid = cid * _NS + sid
        pltpu.async_copy(rows_hbm.at[wid, 0], ridx, semi)
        pltpu.async_copy(cols_hbm.at[wid, 0], cidx, semi)

        @pl.loop(0, ch)
        def _(r):
            @pl.loop(0, d, step=_L)
            def _(c0):
                b0[r, pl.ds(c0, _L)] = jnp.zeros((_L,), jnp.float32)

        @pl.loop(0, npc, step=ch)
        def _(r0):
            pltpu.sync_copy(b0, agg_sh.at[pl.ds(sid * npc + r0, ch)])

        plsc.subcore_barrier()
        pltpu.make_async_copy(rows_hbm.at[wid, 0], ridx, semi).wait()
        pltpu.make_async_copy(cols_hbm.at[wid, 0], cidx, semi).wait()

        @pl.loop(0, nwin)
        def _(w):
            for k in range(nring):
                pltpu.async_copy(y_hbm.at[ridx.at[k]], bufs[k], sems[k])

            @pl.loop(0, main_hi, step=nring)
            def _(j):
                for k in range(nring):
                    pltpu.make_async_copy(y_hbm.at[ridx.at[j + k]], bufs[k],
                                          sems[k]).wait()
                    pltpu.sync_copy(bufs[k], agg_sh.at[cidx.at[j + k]],
                                    add=True)
                    pltpu.async_copy(y_hbm.at[ridx.at[j + k + nring]],
                                     bufs[k], sems[k])

            for k in range(nring):
                pltpu.make_async_copy(y_hbm.at[ridx.at[main_hi + k]],
                                      bufs[k], sems[k]).wait()
                pltpu.sync_copy(bufs[k], agg_sh.at[cidx.at[main_hi + k]],
                                add=True)
                if k < rem:
                    pltpu.async_copy(
                        y_hbm.at[ridx.at[main_hi + nring + k]],
                        bufs[k], sems[k])
            for k in range(rem):
                pltpu.make_async_copy(y_hbm.at[ridx.at[main_hi + nring + k]],
                                      bufs[k], sems[k]).wait()
                pltpu.sync_copy(bufs[k],
                                agg_sh.at[cidx.at[main_hi + nring + k]],
                                add=True)

            @pl.when(w + 1 < nwin)
            def _():
                pltpu.sync_copy(rows_hbm.at[wid, w + 1], ridx)
                pltpu.sync_copy(cols_hbm.at[wid, w + 1], cidx)

        plsc.subcore_barrier()

        @pl.loop(0, npc, step=4 * ch)
        def _(r0):
            for k in range(4):
                pltpu.async_copy(
                    agg_sh.at[pl.ds(sid * npc + r0 + k * ch, ch)],
                    bufs[k], sems[k])
            for k in range(4):
                pltpu.make_async_copy(
                    agg_sh.at[pl.ds(sid * npc + r0 + k * ch, ch)],
                    bufs[k], sems[k]).wait()
                pltpu.async_copy(
                    bufs[k],
                    out_hbm.at[cid, pl.ds(sid * npc + r0 + k * ch, ch)],
                    sems[k])
            for k in range(4):
                pltpu.make_async_copy(
                    bufs[k],
                    out_hbm.at[cid, pl.ds(sid * npc + r0 + k * ch, ch)],
                    sems[k]).wait()

    return agg_kernel(y, rows4, cols4)


_BN = 2000


def _mm_scale_body(x_ref, w_ref, d0_ref, d1_ref, y_ref, dis_ref):
    deg = d0_ref[...] + d1_ref[...] + 1.0
    dis = lax.rsqrt(deg)
    y_ref[...] = jnp.dot(x_ref[...], w_ref[...],
                         preferred_element_type=jnp.float32) * dis
    dis_ref[...] = dis


def _mm_scale(x, w, d0, d1):
    n, din = x.shape
    dout = w.shape[1]
    blk1 = pl.BlockSpec((_BN, 1), lambda i: (i, 0))
    return pl.pallas_call(
        _mm_scale_body,
        grid=(n // _BN,),
        in_specs=[pl.BlockSpec((_BN, din), lambda i: (i, 0)),
                  pl.BlockSpec((din, dout), lambda i: (0, 0)),
                  blk1, blk1],
        out_specs=[pl.BlockSpec((_BN, dout), lambda i: (i, 0)), blk1],
        out_shape=[jax.ShapeDtypeStruct((n, dout), jnp.float32),
                   jax.ShapeDtypeStruct((n, 1), jnp.float32)],
    )(x, w, d0, d1)


def _mid_body(y_ref, p0_ref, p1_ref, dis_ref, b_ref, w_ref, o_ref):
    dis = dis_ref[...]
    h = (y_ref[...] + p0_ref[0] + p1_ref[0]) * dis + b_ref[...]
    o_ref[...] = jnp.dot(h, w_ref[...],
                         preferred_element_type=jnp.float32) * dis


def _mid(y, p, dis, b, w):
    n, d = y.shape
    dout = w.shape[1]
    blk2 = pl.BlockSpec((_BN, d), lambda i: (i, 0))
    return pl.pallas_call(
        _mid_body,
        grid=(n // _BN,),
        in_specs=[blk2,
                  pl.BlockSpec((1, _BN, d), lambda i: (0, i, 0)),
                  pl.BlockSpec((1, _BN, d), lambda i: (1, i, 0)),
                  pl.BlockSpec((_BN, 1), lambda i: (i, 0)),
                  pl.BlockSpec((1, d), lambda i: (0, 0)),
                  pl.BlockSpec((d, dout), lambda i: (0, 0))],
        out_specs=pl.BlockSpec((_BN, dout), lambda i: (i, 0)),
        out_shape=jax.ShapeDtypeStruct((n, dout), jnp.float32),
    )(y, p, p, dis, b, w)


def _final_body(y_ref, q0_ref, q1_ref, dis_ref, b_ref, o_ref):
    o_ref[...] = (y_ref[...] + q0_ref[0] + q1_ref[0]) * dis_ref[...] \
        + b_ref[...]


def _final(y, q, dis, b):
    n, d = y.shape
    blk2 = pl.BlockSpec((_BN, d), lambda i: (i, 0))
    return pl.pallas_call(
        _final_body,
        grid=(n // _BN,),
        in_specs=[blk2,
                  pl.BlockSpec((1, _BN, d), lambda i: (0, i, 0)),
                  pl.BlockSpec((1, _BN, d), lambda i: (1, i, 0)),
                  pl.BlockSpec((_BN, 1), lambda i: (i, 0)),
                  pl.BlockSpec((1, d), lambda i: (0, 0))],
        out_specs=pl.BlockSpec((_BN, d), lambda i: (i, 0)),
        out_shape=jax.ShapeDtypeStruct((n, d), jnp.float32),
    )(y, q, q, dis, b)


def kernel(x, edge_index, W1, b1, W2, b2):
    n, _ = x.shape
    e = edge_index.shape[1]
    epw = e // _NW
    ch = 80
    nwin = 5
    wchunk = epw // ch // nwin
    rows4 = edge_index[0].reshape(_NW, nwin, wchunk, ch)
    cols4 = edge_index[1].reshape(_NW, nwin, wchunk, ch)
    n_pad = -(-n // (_NS * 8)) * (_NS * 8)
    n_acc = -(-n // (_NS * 128)) * (_NS * 128)

    degp = _degree_partials(
        edge_index[1].reshape(_NW, 5, epw // 80 // 5, 80),
        n_pad).reshape(_NC, n_pad)
    d0 = degp[0, :n].reshape(n, 1)
    d1 = degp[1, :n].reshape(n, 1)
    y1, dis = _mm_scale(x, W1, d0, d1)
    p = _aggregate(y1, rows4, cols4, n_acc)
    y2 = _mid(y1, p, dis, b1.reshape(1, -1), W2)
    q = _aggregate(y2, rows4, cols4, n_acc)
    return _final(y2, q, dis, b2.reshape(1, -1))


# speedup vs baseline: 1.0697x; 1.0697x over previous
import functools

import jax
import jax.numpy as jnp
from jax import lax
from jax.experimental import pallas as pl
from jax.experimental.pallas import tpu as pltpu
from jax.experimental.pallas import tpu_sc as plsc

_NC = 2
_NS = 16
_L = 16
_NW = _NC * _NS

_MESH = dict(core_axis_name="c", subcore_axis_name="s")


def _degree_partials(ei5, n_pad):
    _, nw, nwin, wchunk, ch = ei5.shape
    zps = n_pad // _NS

    @functools.partial(
        pl.kernel,
        out_type=jax.ShapeDtypeStruct((_NC * n_pad,), jnp.float32),
        mesh=plsc.VectorSubcoreMesh(**_MESH),
        scratch_types=[
            pltpu.VMEM((wchunk, ch), jnp.int32),
            pltpu.VMEM((ch,), jnp.float32),
            pltpu.VMEM((zps,), jnp.float32),
            pltpu.VMEM_SHARED((n_pad,), jnp.float32),
            pltpu.SemaphoreType.DMA,
        ],
    )
    def deg_kernel(ei_hbm, out_hbm, cidx, ones_v, zeros_v, deg_sh, sem):
        cid = lax.axis_index("c")
        sid = lax.axis_index("s")
        wid = cid * _NS + sid

        @pl.loop(0, ch, step=_L)
        def _(i):
            ones_v[pl.ds(i, _L)] = jnp.ones((_L,), jnp.float32)

        @pl.loop(0, zps, step=_L)
        def _(i):
            zeros_v[pl.ds(i, _L)] = jnp.zeros((_L,), jnp.float32)

        pltpu.sync_copy(zeros_v, deg_sh.at[pl.ds(sid * zps, zps)])
        plsc.subcore_barrier()

        @pl.loop(0, nwin)
        def _(w):
            pltpu.sync_copy(ei_hbm.at[1, wid, w], cidx)

            @pl.loop(0, wchunk)
            def _(j):
                pltpu.sync_copy(ones_v, deg_sh.at[cidx.at[j]], add=True)

        plsc.subcore_barrier()
        pltpu.sync_copy(deg_sh.at[pl.ds(sid * zps, zps)], zeros_v)
        pltpu.sync_copy(zeros_v, out_hbm.at[pl.ds(cid * n_pad + sid * zps, zps)])

    return deg_kernel(ei5)


def _aggregate(y, ei5):
    n_acc, d = y.shape
    _, nw, nwin, wchunk, ch = ei5.shape
    npc = n_acc // _NS
    nring = 4
    rem = wchunk % nring
    main_hi = wchunk - nring - rem

    @functools.partial(
        pl.kernel,
        out_type=jax.ShapeDtypeStruct((_NC, n_acc, d), jnp.float32),
        mesh=plsc.VectorSubcoreMesh(**_MESH),
        scratch_types=[
            pltpu.VMEM((wchunk, ch), jnp.int32),
            pltpu.VMEM((wchunk, ch), jnp.int32),
            pltpu.VMEM((ch, d), jnp.float32),
            pltpu.VMEM((ch, d), jnp.float32),
            pltpu.VMEM((ch, d), jnp.float32),
            pltpu.VMEM((ch, d), jnp.float32),
            pltpu.VMEM_SHARED((n_acc, d), jnp.float32),
            pltpu.SemaphoreType.DMA,
            pltpu.SemaphoreType.DMA,
            pltpu.SemaphoreType.DMA,
            pltpu.SemaphoreType.DMA,
            pltpu.SemaphoreType.DMA,
        ],
    )
    def agg_kernel(y_hbm, ei_hbm, out_hbm,
                   ridx, cidx, b0, b1, b2, b3, agg_sh,
                   semi, s0, s1, s2, s3):
        bufs = (b0, b1, b2, b3)
        sems = (s0, s1, s2, s3)
        cid = lax.axis_index("c")
        sid = lax.axis_index("s")
        wid = cid * _NS + sid
        pltpu.async_copy(ei_hbm.at[0, wid, 0], ridx, semi)
        pltpu.async_copy(ei_hbm.at[1, wid, 0], cidx, semi)

        @pl.loop(0, ch)
        def _(r):
            @pl.loop(0, d, step=_L)
            def _(c0):
                b0[r, pl.ds(c0, _L)] = jnp.zeros((_L,), jnp.float32)

        @pl.loop(0, npc, step=ch)
        def _(r0):
            pltpu.sync_copy(b0, agg_sh.at[pl.ds(sid * npc + r0, ch)])

        plsc.subcore_barrier()
        pltpu.make_async_copy(ei_hbm.at[0, wid, 0], ridx, semi).wait()
        pltpu.make_async_copy(ei_hbm.at[1, wid, 0], cidx, semi).wait()

        @pl.loop(0, nwin)
        def _(w):
            for k in range(nring):
                pltpu.async_copy(y_hbm.at[ridx.at[k]], bufs[k], sems[k])

            @pl.loop(0, main_hi, step=nring)
            def _(j):
                for k in range(nring):
                    pltpu.make_async_copy(y_hbm.at[ridx.at[j + k]], bufs[k],
                                          sems[k]).wait()
                    pltpu.sync_copy(bufs[k], agg_sh.at[cidx.at[j + k]],
                                    add=True)
                    pltpu.async_copy(y_hbm.at[ridx.at[j + k + nring]],
                                     bufs[k], sems[k])

            for k in range(nring):
                pltpu.make_async_copy(y_hbm.at[ridx.at[main_hi + k]],
                                      bufs[k], sems[k]).wait()
                pltpu.sync_copy(bufs[k], agg_sh.at[cidx.at[main_hi + k]],
                                add=True)
                if k < rem:
                    pltpu.async_copy(
                        y_hbm.at[ridx.at[main_hi + nring + k]],
                        bufs[k], sems[k])
            for k in range(rem):
                pltpu.make_async_copy(y_hbm.at[ridx.at[main_hi + nring + k]],
                                      bufs[k], sems[k]).wait()
                pltpu.sync_copy(bufs[k],
                                agg_sh.at[cidx.at[main_hi + nring + k]],
                                add=True)

            @pl.when(w + 1 < nwin)
            def _():
                pltpu.sync_copy(ei_hbm.at[0, wid, w + 1], ridx)
                pltpu.sync_copy(ei_hbm.at[1, wid, w + 1], cidx)

        plsc.subcore_barrier()

        @pl.loop(0, npc, step=4 * ch)
        def _(r0):
            for k in range(4):
                pltpu.async_copy(
                    agg_sh.at[pl.ds(sid * npc + r0 + k * ch, ch)],
                    bufs[k], sems[k])
            for k in range(4):
                pltpu.make_async_copy(
                    agg_sh.at[pl.ds(sid * npc + r0 + k * ch, ch)],
                    bufs[k], sems[k]).wait()
                pltpu.async_copy(
                    bufs[k],
                    out_hbm.at[cid, pl.ds(sid * npc + r0 + k * ch, ch)],
                    sems[k])
            for k in range(4):
                pltpu.make_async_copy(
                    bufs[k],
                    out_hbm.at[cid, pl.ds(sid * npc + r0 + k * ch, ch)],
                    sems[k]).wait()

    return agg_kernel(y, ei5)


_BN = 2048


def _mm_scale_body(x_ref, w_ref, dp_ref, y_ref, dis_ref):
    i = pl.program_id(0)
    deg = dp_ref[0, pl.ds(i * _BN, _BN)] + dp_ref[1, pl.ds(i * _BN, _BN)] + 1.0
    dis = lax.rsqrt(deg)[:, None]
    y_ref[...] = jnp.dot(x_ref[...], w_ref[...],
                         preferred_element_type=jnp.float32) * dis
    dis_ref[...] = dis


def _mm_scale(x, w, degp, n_acc):
    n, din = x.shape
    dout = w.shape[1]
    return pl.pallas_call(
        _mm_scale_body,
        grid=(n_acc // _BN,),
        in_specs=[pl.BlockSpec((_BN, din), lambda i: (i, 0)),
                  pl.BlockSpec((din, dout), lambda i: (0, 0)),
                  pl.BlockSpec((2, n_acc), lambda i: (0, 0))],
        out_specs=[pl.BlockSpec((_BN, dout), lambda i: (i, 0)),
                   pl.BlockSpec((_BN, 1), lambda i: (i, 0))],
        out_shape=[jax.ShapeDtypeStruct((n_acc, dout), jnp.float32),
                   jax.ShapeDtypeStruct((n_acc, 1), jnp.float32)],
    )(x, w, degp)


def _mid_body(y_ref, p0_ref, p1_ref, dis_ref, b_ref, w_ref, o_ref):
    dis = dis_ref[...]
    h = (y_ref[...] + p0_ref[0] + p1_ref[0]) * dis + b_ref[...]
    o_ref[...] = jnp.dot(h, w_ref[...],
                         preferred_element_type=jnp.float32) * dis


def _mid(y, p, dis, b, w):
    n_acc, d = y.shape
    dout = w.shape[1]
    blk2 = pl.BlockSpec((_BN, d), lambda i: (i, 0))
    return pl.pallas_call(
        _mid_body,
        grid=(n_acc // _BN,),
        in_specs=[blk2,
                  pl.BlockSpec((1, _BN, d), lambda i: (0, i, 0)),
                  pl.BlockSpec((1, _BN, d), lambda i: (1, i, 0)),
                  pl.BlockSpec((_BN, 1), lambda i: (i, 0)),
                  pl.BlockSpec((1, d), lambda i: (0, 0)),
                  pl.BlockSpec((d, dout), lambda i: (0, 0))],
        out_specs=pl.BlockSpec((_BN, dout), lambda i: (i, 0)),
        out_shape=jax.ShapeDtypeStruct((n_acc, dout), jnp.float32),
    )(y, p, p, dis, b, w)


_BNF = 2000


def _final_body(y_ref, q0_ref, q1_ref, dis_ref, b_ref, o_ref):
    o_ref[...] = (y_ref[...] + q0_ref[0] + q1_ref[0]) * dis_ref[...] \
        + b_ref[...]


def _final(y, q, dis, b, n):
    n_acc, d = y.shape
    blk2 = pl.BlockSpec((_BNF, d), lambda i: (i, 0))
    return pl.pallas_call(
        _final_body,
        grid=(n // _BNF,),
        in_specs=[blk2,
                  pl.BlockSpec((1, _BNF, d), lambda i: (0, i, 0)),
                  pl.BlockSpec((1, _BNF, d), lambda i: (1, i, 0)),
                  pl.BlockSpec((_BNF, 1), lambda i: (i, 0)),
                  pl.BlockSpec((1, d), lambda i: (0, 0))],
        out_specs=pl.BlockSpec((_BNF, d), lambda i: (i, 0)),
        out_shape=jax.ShapeDtypeStruct((n, d), jnp.float32),
    )(y, q, q, dis, b)


def kernel(x, edge_index, W1, b1, W2, b2):
    n, _ = x.shape
    e = edge_index.shape[1]
    epw = e // _NW
    ch = 80
    nwin = 5
    wchunk = epw // ch // nwin
    ei5 = edge_index.reshape(2, _NW, nwin, wchunk, ch)
    n_acc = -(-n // (_NS * 128)) * (_NS * 128)

    degp = _degree_partials(ei5, n_acc).reshape(_NC, n_acc)
    y1, dis = _mm_scale(x, W1, degp, n_acc)
    p = _aggregate(y1, ei5)
    y2 = _mid(y1, p, dis, b1.reshape(1, -1), W2)
    q = _aggregate(y2, ei5)
    return _final(y2, q, dis, b2.reshape(1, -1), n)


# speedup vs baseline: 1.0759x; 1.0058x over previous
import functools

import jax
import jax.numpy as jnp
from jax import lax
from jax.experimental import pallas as pl
from jax.experimental.pallas import tpu as pltpu
from jax.experimental.pallas import tpu_sc as plsc

_NC = 2
_NS = 16
_L = 16
_NW = _NC * _NS

_MESH = dict(core_axis_name="c", subcore_axis_name="s")


def _degree_partials(ei5, n_pad):
    _, nw, nwin, wchunk, ch = ei5.shape
    zps = n_pad // _NS

    @functools.partial(
        pl.kernel,
        out_type=jax.ShapeDtypeStruct((_NC * n_pad,), jnp.float32),
        mesh=plsc.VectorSubcoreMesh(**_MESH),
        scratch_types=[
            pltpu.VMEM((wchunk, ch), jnp.int32),
            pltpu.VMEM((ch,), jnp.float32),
            pltpu.VMEM((zps,), jnp.float32),
            pltpu.VMEM_SHARED((n_pad,), jnp.float32),
            pltpu.SemaphoreType.DMA,
        ],
    )
    def deg_kernel(ei_hbm, out_hbm, cidx, ones_v, zeros_v, deg_sh, sem):
        cid = lax.axis_index("c")
        sid = lax.axis_index("s")
        wid = cid * _NS + sid

        @pl.loop(0, ch, step=_L)
        def _(i):
            ones_v[pl.ds(i, _L)] = jnp.ones((_L,), jnp.float32)

        @pl.loop(0, zps, step=_L)
        def _(i):
            zeros_v[pl.ds(i, _L)] = jnp.zeros((_L,), jnp.float32)

        pltpu.sync_copy(zeros_v, deg_sh.at[pl.ds(sid * zps, zps)])
        plsc.subcore_barrier()

        @pl.loop(0, nwin)
        def _(w):
            pltpu.sync_copy(ei_hbm.at[1, wid, w], cidx)

            @pl.loop(0, wchunk)
            def _(j):
                pltpu.sync_copy(ones_v, deg_sh.at[cidx.at[j]], add=True)

        plsc.subcore_barrier()
        pltpu.sync_copy(deg_sh.at[pl.ds(sid * zps, zps)], zeros_v)
        pltpu.sync_copy(zeros_v, out_hbm.at[pl.ds(cid * n_pad + sid * zps, zps)])

    return deg_kernel(ei5)


def _aggregate(y, ei5):
    n_acc, d = y.shape
    _, nw, nwin, wchunk, ch = ei5.shape
    npc = n_acc // _NS
    nring = 4
    rem = wchunk % nring
    main_hi = wchunk - nring - rem

    @functools.partial(
        pl.kernel,
        out_type=jax.ShapeDtypeStruct((_NC, n_acc, d), jnp.float32),
        mesh=plsc.VectorSubcoreMesh(**_MESH),
        scratch_types=[
            pltpu.VMEM((wchunk, ch), jnp.int32),
            pltpu.VMEM((wchunk, ch), jnp.int32),
            pltpu.VMEM((ch, d), jnp.float32),
            pltpu.VMEM((ch, d), jnp.float32),
            pltpu.VMEM((ch, d), jnp.float32),
            pltpu.VMEM((ch, d), jnp.float32),
            pltpu.VMEM_SHARED((n_acc, d), jnp.float32),
            pltpu.SemaphoreType.DMA,
            pltpu.SemaphoreType.DMA,
            pltpu.SemaphoreType.DMA,
            pltpu.SemaphoreType.DMA,
            pltpu.SemaphoreType.DMA,
        ],
    )
    def agg_kernel(y_hbm, ei_hbm, out_hbm,
                   ridx, cidx, b0, b1, b2, b3, agg_sh,
                   semi, s0, s1, s2, s3):
        bufs = (b0, b1, b2, b3)
        sems = (s0, s1, s2, s3)
        cid = lax.axis_index("c")
        sid = lax.axis_index("s")
        wid = cid * _NS + sid
        pltpu.async_copy(ei_hbm.at[0, wid, 0], ridx, semi)
        pltpu.async_copy(ei_hbm.at[1, wid, 0], cidx, semi)

        @pl.loop(0, ch)
        def _(r):
            @pl.loop(0, d, step=_L)
            def _(c0):
                b0[r, pl.ds(c0, _L)] = jnp.zeros((_L,), jnp.float32)

        @pl.loop(0, npc, step=ch)
        def _(r0):
            pltpu.sync_copy(b0, agg_sh.at[pl.ds(sid * npc + r0, ch)])

        plsc.subcore_barrier()
        pltpu.make_async_copy(ei_hbm.at[0, wid, 0], ridx, semi).wait()
        pltpu.make_async_copy(ei_hbm.at[1, wid, 0], cidx, semi).wait()

        @pl.loop(0, nwin)
        def _(w):
            for k in range(nring):
                pltpu.async_copy(y_hbm.at[ridx.at[k]], bufs[k], sems[k])

            @pl.loop(0, main_hi, step=nring)
            def _(j):
                for k in range(nring):
                    pltpu.make_async_copy(y_hbm.at[ridx.at[j + k]], bufs[k],
                                          sems[k]).wait()
                    pltpu.sync_copy(bufs[k], agg_sh.at[cidx.at[j + k]],
                                    add=True)
                    pltpu.async_copy(y_hbm.at[ridx.at[j + k + nring]],
                                     bufs[k], sems[k])

            for k in range(nring):
                pltpu.make_async_copy(y_hbm.at[ridx.at[main_hi + k]],
                                      bufs[k], sems[k]).wait()
                pltpu.sync_copy(bufs[k], agg_sh.at[cidx.at[main_hi + k]],
                                add=True)
                if k < rem:
                    pltpu.async_copy(
                        y_hbm.at[ridx.at[main_hi + nring + k]],
                        bufs[k], sems[k])
            for k in range(rem):
                pltpu.make_async_copy(y_hbm.at[ridx.at[main_hi + nring + k]],
                                      bufs[k], sems[k]).wait()
                pltpu.sync_copy(bufs[k],
                                agg_sh.at[cidx.at[main_hi + nring + k]],
                                add=True)

            @pl.when(w + 1 < nwin)
            def _():
                pltpu.sync_copy(ei_hbm.at[0, wid, w + 1], ridx)
                pltpu.sync_copy(ei_hbm.at[1, wid, w + 1], cidx)

        plsc.subcore_barrier()

        @pl.loop(0, npc, step=4 * ch)
        def _(r0):
            for k in range(4):
                pltpu.async_copy(
                    agg_sh.at[pl.ds(sid * npc + r0 + k * ch, ch)],
                    bufs[k], sems[k])
            for k in range(4):
                pltpu.make_async_copy(
                    agg_sh.at[pl.ds(sid * npc + r0 + k * ch, ch)],
                    bufs[k], sems[k]).wait()
                pltpu.async_copy(
                    bufs[k],
                    out_hbm.at[cid, pl.ds(sid * npc + r0 + k * ch, ch)],
                    sems[k])
            for k in range(4):
                pltpu.make_async_copy(
                    bufs[k],
                    out_hbm.at[cid, pl.ds(sid * npc + r0 + k * ch, ch)],
                    sems[k]).wait()

    return agg_kernel(y, ei5)


_BN = 2048


def _mm_scale_body(x_ref, w_ref, dp_ref, y_ref, dis_ref):
    i = pl.program_id(0)
    n_acc = dp_ref.shape[0] // 2
    deg = (dp_ref[pl.ds(i * _BN, _BN)]
           + dp_ref[pl.ds(n_acc + i * _BN, _BN)] + 1.0)
    dis = lax.rsqrt(deg)[:, None]
    y_ref[...] = jnp.dot(x_ref[...], w_ref[...],
                         preferred_element_type=jnp.float32) * dis
    dis_ref[...] = dis


def _mm_scale(x, w, degp, n_acc):
    n, din = x.shape
    dout = w.shape[1]
    return pl.pallas_call(
        _mm_scale_body,
        grid=(n_acc // _BN,),
        in_specs=[pl.BlockSpec((_BN, din), lambda i: (i, 0)),
                  pl.BlockSpec((din, dout), lambda i: (0, 0)),
                  pl.BlockSpec((2 * n_acc,), lambda i: (0,))],
        out_specs=[pl.BlockSpec((_BN, dout), lambda i: (i, 0)),
                   pl.BlockSpec((_BN, 1), lambda i: (i, 0))],
        out_shape=[jax.ShapeDtypeStruct((n_acc, dout), jnp.float32),
                   jax.ShapeDtypeStruct((n_acc, 1), jnp.float32)],
    )(x, w, degp)


def _mid_body(y_ref, p0_ref, p1_ref, dis_ref, b_ref, w_ref, o_ref):
    dis = dis_ref[...]
    h = (y_ref[...] + p0_ref[0] + p1_ref[0]) * dis + b_ref[...]
    o_ref[...] = jnp.dot(h, w_ref[...],
                         preferred_element_type=jnp.float32) * dis


def _mid(y, p, dis, b, w):
    n_acc, d = y.shape
    dout = w.shape[1]
    blk2 = pl.BlockSpec((_BN, d), lambda i: (i, 0))
    return pl.pallas_call(
        _mid_body,
        grid=(n_acc // _BN,),
        in_specs=[blk2,
                  pl.BlockSpec((1, _BN, d), lambda i: (0, i, 0)),
                  pl.BlockSpec((1, _BN, d), lambda i: (1, i, 0)),
                  pl.BlockSpec((_BN, 1), lambda i: (i, 0)),
                  pl.BlockSpec((1, d), lambda i: (0, 0)),
                  pl.BlockSpec((d, dout), lambda i: (0, 0))],
        out_specs=pl.BlockSpec((_BN, dout), lambda i: (i, 0)),
        out_shape=jax.ShapeDtypeStruct((n_acc, dout), jnp.float32),
    )(y, p, p, dis, b, w)


_BNF = 2000


def _final_body(y_ref, q0_ref, q1_ref, dis_ref, b_ref, o_ref):
    o_ref[...] = (y_ref[...] + q0_ref[0] + q1_ref[0]) * dis_ref[...] \
        + b_ref[...]


def _final(y, q, dis, b, n):
    n_acc, d = y.shape
    blk2 = pl.BlockSpec((_BNF, d), lambda i: (i, 0))
    return pl.pallas_call(
        _final_body,
        grid=(n // _BNF,),
        in_specs=[blk2,
                  pl.BlockSpec((1, _BNF, d), lambda i: (0, i, 0)),
                  pl.BlockSpec((1, _BNF, d), lambda i: (1, i, 0)),
                  pl.BlockSpec((_BNF, 1), lambda i: (i, 0)),
                  pl.BlockSpec((1, d), lambda i: (0, 0))],
        out_specs=pl.BlockSpec((_BNF, d), lambda i: (i, 0)),
        out_shape=jax.ShapeDtypeStruct((n, d), jnp.float32),
    )(y, q, q, dis, b)


def kernel(x, edge_index, W1, b1, W2, b2):
    n, _ = x.shape
    e = edge_index.shape[1]
    epw = e // _NW
    ch = 80
    nwin = 5
    wchunk = epw // ch // nwin
    ei5 = edge_index.reshape(2, _NW, nwin, wchunk, ch)
    n_acc = -(-n // (_NS * 128)) * (_NS * 128)

    degp = _degree_partials(ei5, n_acc)
    y1, dis = _mm_scale(x, W1, degp, n_acc)
    p = _aggregate(y1, ei5)
    y2 = _mid(y1, p, dis, b1.reshape(1, -1), W2)
    q = _aggregate(y2, ei5)
    return _final(y2, q, dis, b2.reshape(1, -1), n)


# speedup vs baseline: 1.1165x; 1.0377x over previous
import functools

import jax
import jax.numpy as jnp
from jax import lax
from jax.experimental import pallas as pl
from jax.experimental.pallas import tpu as pltpu
from jax.experimental.pallas import tpu_sc as plsc

_NC = 2
_NS = 16
_L = 16
_NW = _NC * _NS

_MESH = dict(core_axis_name="c", subcore_axis_name="s")


def _degree_partials(ei5, n_pad):
    _, nw, nwin, wchunk, ch = ei5.shape
    zps = n_pad // _NS

    @functools.partial(
        pl.kernel,
        out_type=jax.ShapeDtypeStruct((_NC * n_pad,), jnp.float32),
        mesh=plsc.VectorSubcoreMesh(**_MESH),
        scratch_types=[
            pltpu.VMEM((wchunk, ch), jnp.int32),
            pltpu.VMEM((wchunk, ch), jnp.int32),
            pltpu.VMEM((ch,), jnp.float32),
            pltpu.VMEM((zps,), jnp.float32),
            pltpu.VMEM_SHARED((n_pad,), jnp.float32),
            pltpu.SemaphoreType.DMA,
            pltpu.SemaphoreType.DMA,
        ],
    )
    def deg_kernel(ei_hbm, out_hbm, cid0, cid1, ones_v, zeros_v, deg_sh,
                   semi, sems):
        cid = lax.axis_index("c")
        sid = lax.axis_index("s")
        wid = cid * _NS + sid
        cbufs = (cid0, cid1)
        pltpu.async_copy(ei_hbm.at[1, wid, 0], cid0, semi)

        @pl.loop(0, ch, step=_L)
        def _(i):
            ones_v[pl.ds(i, _L)] = jnp.ones((_L,), jnp.float32)

        @pl.loop(0, zps, step=_L)
        def _(i):
            zeros_v[pl.ds(i, _L)] = jnp.zeros((_L,), jnp.float32)

        pltpu.sync_copy(zeros_v, deg_sh.at[pl.ds(sid * zps, zps)])
        plsc.subcore_barrier()
        pltpu.make_async_copy(ei_hbm.at[1, wid, 0], cid0, semi).wait()

        for w in range(nwin):
            cbuf = cbufs[w % 2]
            if w + 1 < nwin:
                pltpu.async_copy(ei_hbm.at[1, wid, w + 1],
                                 cbufs[(w + 1) % 2], semi)

            @pl.loop(0, wchunk)
            def _(j):
                pltpu.async_copy(ones_v, deg_sh.at[cbuf.at[j]], sems,
                                 add=True)

            @pl.loop(0, wchunk)
            def _(j):
                pltpu.make_async_copy(ones_v, deg_sh.at[cbuf.at[j]],
                                      sems).wait()

            if w + 1 < nwin:
                pltpu.make_async_copy(ei_hbm.at[1, wid, w + 1],
                                      cbufs[(w + 1) % 2], semi).wait()

        plsc.subcore_barrier()
        pltpu.sync_copy(deg_sh.at[pl.ds(sid * zps, zps)], zeros_v)
        pltpu.sync_copy(zeros_v, out_hbm.at[pl.ds(cid * n_pad + sid * zps, zps)])

    return deg_kernel(ei5)


def _aggregate(y, ei5):
    n_acc, d = y.shape
    _, nw, nwin, wchunk, ch = ei5.shape
    npc = n_acc // _NS
    nring = 4
    rem = wchunk % nring
    main_hi = wchunk - nring - rem

    @functools.partial(
        pl.kernel,
        out_type=jax.ShapeDtypeStruct((_NC, n_acc, d), jnp.float32),
        mesh=plsc.VectorSubcoreMesh(**_MESH),
        scratch_types=[
            pltpu.VMEM((wchunk, ch), jnp.int32),
            pltpu.VMEM((wchunk, ch), jnp.int32),
            pltpu.VMEM((ch, d), jnp.float32),
            pltpu.VMEM((ch, d), jnp.float32),
            pltpu.VMEM((ch, d), jnp.float32),
            pltpu.VMEM((ch, d), jnp.float32),
            pltpu.VMEM_SHARED((n_acc, d), jnp.float32),
            pltpu.SemaphoreType.DMA,
            pltpu.SemaphoreType.DMA,
            pltpu.SemaphoreType.DMA,
            pltpu.SemaphoreType.DMA,
            pltpu.SemaphoreType.DMA,
        ],
    )
    def agg_kernel(y_hbm, ei_hbm, out_hbm,
                   ridx, cidx, b0, b1, b2, b3, agg_sh,
                   semi, s0, s1, s2, s3):
        bufs = (b0, b1, b2, b3)
        sems = (s0, s1, s2, s3)
        cid = lax.axis_index("c")
        sid = lax.axis_index("s")
        wid = cid * _NS + sid
        pltpu.async_copy(ei_hbm.at[0, wid, 0], ridx, semi)
        pltpu.async_copy(ei_hbm.at[1, wid, 0], cidx, semi)

        @pl.loop(0, ch)
        def _(r):
            @pl.loop(0, d, step=_L)
            def _(c0):
                b0[r, pl.ds(c0, _L)] = jnp.zeros((_L,), jnp.float32)

        @pl.loop(0, npc, step=ch)
        def _(r0):
            pltpu.sync_copy(b0, agg_sh.at[pl.ds(sid * npc + r0, ch)])

        plsc.subcore_barrier()
        pltpu.make_async_copy(ei_hbm.at[0, wid, 0], ridx, semi).wait()
        pltpu.make_async_copy(ei_hbm.at[1, wid, 0], cidx, semi).wait()

        @pl.loop(0, nwin)
        def _(w):
            for k in range(nring):
                pltpu.async_copy(y_hbm.at[ridx.at[k]], bufs[k], sems[k])

            @pl.loop(0, main_hi, step=nring)
            def _(j):
                for k in range(nring):
                    pltpu.make_async_copy(y_hbm.at[ridx.at[j + k]], bufs[k],
                                          sems[k]).wait()
                    pltpu.sync_copy(bufs[k], agg_sh.at[cidx.at[j + k]],
                                    add=True)
                    pltpu.async_copy(y_hbm.at[ridx.at[j + k + nring]],
                                     bufs[k], sems[k])

            for k in range(nring):
                pltpu.make_async_copy(y_hbm.at[ridx.at[main_hi + k]],
                                      bufs[k], sems[k]).wait()
                pltpu.sync_copy(bufs[k], agg_sh.at[cidx.at[main_hi + k]],
                                add=True)
                if k < rem:
                    pltpu.async_copy(
                        y_hbm.at[ridx.at[main_hi + nring + k]],
                        bufs[k], sems[k])
            for k in range(rem):
                pltpu.make_async_copy(y_hbm.at[ridx.at[main_hi + nring + k]],
                                      bufs[k], sems[k]).wait()
                pltpu.sync_copy(bufs[k],
                                agg_sh.at[cidx.at[main_hi + nring + k]],
                                add=True)

            @pl.when(w + 1 < nwin)
            def _():
                pltpu.sync_copy(ei_hbm.at[0, wid, w + 1], ridx)
                pltpu.sync_copy(ei_hbm.at[1, wid, w + 1], cidx)

        plsc.subcore_barrier()

        @pl.loop(0, npc, step=4 * ch)
        def _(r0):
            for k in range(4):
                pltpu.async_copy(
                    agg_sh.at[pl.ds(sid * npc + r0 + k * ch, ch)],
                    bufs[k], sems[k])
            for k in range(4):
                pltpu.make_async_copy(
                    agg_sh.at[pl.ds(sid * npc + r0 + k * ch, ch)],
                    bufs[k], sems[k]).wait()
                pltpu.async_copy(
                    bufs[k],
                    out_hbm.at[cid, pl.ds(sid * npc + r0 + k * ch, ch)],
                    sems[k])
            for k in range(4):
                pltpu.make_async_copy(
                    bufs[k],
                    out_hbm.at[cid, pl.ds(sid * npc + r0 + k * ch, ch)],
                    sems[k]).wait()

    return agg_kernel(y, ei5)


_BN = 2048


def _mm_scale_body(x_ref, w_ref, dp_ref, y_ref, dis_ref):
    i = pl.program_id(0)
    n_acc = dp_ref.shape[0] // 2
    deg = (dp_ref[pl.ds(i * _BN, _BN)]
           + dp_ref[pl.ds(n_acc + i * _BN, _BN)] + 1.0)
    dis = lax.rsqrt(deg)[:, None]
    y_ref[...] = jnp.dot(x_ref[...].astype(jnp.bfloat16),
                         w_ref[...].astype(jnp.bfloat16),
                         preferred_element_type=jnp.float32) * dis
    dis_ref[...] = dis


def _mm_scale(x, w, degp, n_acc):
    n, din = x.shape
    dout = w.shape[1]
    return pl.pallas_call(
        _mm_scale_body,
        grid=(n_acc // _BN,),
        in_specs=[pl.BlockSpec((_BN, din), lambda i: (i, 0)),
                  pl.BlockSpec((din, dout), lambda i: (0, 0)),
                  pl.BlockSpec((2 * n_acc,), lambda i: (0,))],
        out_specs=[pl.BlockSpec((_BN, dout), lambda i: (i, 0)),
                   pl.BlockSpec((_BN, 1), lambda i: (i, 0))],
        out_shape=[jax.ShapeDtypeStruct((n_acc, dout), jnp.float32),
                   jax.ShapeDtypeStruct((n_acc, 1), jnp.float32)],
    )(x, w, degp)


def _mid_body(y_ref, p0_ref, p1_ref, dis_ref, b_ref, w_ref, o_ref):
    dis = dis_ref[...]
    h = (y_ref[...] + p0_ref[0] + p1_ref[0]) * dis + b_ref[...]
    o_ref[...] = jnp.dot(h.astype(jnp.bfloat16),
                         w_ref[...].astype(jnp.bfloat16),
                         preferred_element_type=jnp.float32) * dis


def _mid(y, p, dis, b, w):
    n_acc, d = y.shape
    dout = w.shape[1]
    blk2 = pl.BlockSpec((_BN, d), lambda i: (i, 0))
    return pl.pallas_call(
        _mid_body,
        grid=(n_acc // _BN,),
        in_specs=[blk2,
                  pl.BlockSpec((1, _BN, d), lambda i: (0, i, 0)),
                  pl.BlockSpec((1, _BN, d), lambda i: (1, i, 0)),
                  pl.BlockSpec((_BN, 1), lambda i: (i, 0)),
                  pl.BlockSpec((1, d), lambda i: (0, 0)),
                  pl.BlockSpec((d, dout), lambda i: (0, 0))],
        out_specs=pl.BlockSpec((_BN, dout), lambda i: (i, 0)),
        out_shape=jax.ShapeDtypeStruct((n_acc, dout), jnp.float32),
    )(y, p, p, dis, b, w)


_BNF = 2000


def _final_body(y_ref, q0_ref, q1_ref, dis_ref, b_ref, o_ref):
    o_ref[...] = (y_ref[...] + q0_ref[0] + q1_ref[0]) * dis_ref[...] \
        + b_ref[...]


def _final(y, q, dis, b, n):
    n_acc, d = y.shape
    blk2 = pl.BlockSpec((_BNF, d), lambda i: (i, 0))
    return pl.pallas_call(
        _final_body,
        grid=(n // _BNF,),
        in_specs=[blk2,
                  pl.BlockSpec((1, _BNF, d), lambda i: (0, i, 0)),
                  pl.BlockSpec((1, _BNF, d), lambda i: (1, i, 0)),
                  pl.BlockSpec((_BNF, 1), lambda i: (i, 0)),
                  pl.BlockSpec((1, d), lambda i: (0, 0))],
        out_specs=pl.BlockSpec((_BNF, d), lambda i: (i, 0)),
        out_shape=jax.ShapeDtypeStruct((n, d), jnp.float32),
    )(y, q, q, dis, b)


def kernel(x, edge_index, W1, b1, W2, b2):
    n, _ = x.shape
    e = edge_index.shape[1]
    epw = e // _NW
    ch = 80
    nwin = 5
    wchunk = epw // ch // nwin
    ei5 = edge_index.reshape(2, _NW, nwin, wchunk, ch)
    n_acc = -(-n // (_NS * 128)) * (_NS * 128)

    degp = _degree_partials(ei5, n_acc)
    y1, dis = _mm_scale(x, W1, degp, n_acc)
    p = _aggregate(y1, ei5)
    y2 = _mid(y1, p, dis, b1.reshape(1, -1), W2)
    q = _aggregate(y2, ei5)
    return _final(y2, q, dis, b2.reshape(1, -1), n)


# speedup vs baseline: 1.1207x; 1.0038x over previous
import functools

import jax
import jax.numpy as jnp
from jax import lax
from jax.experimental import pallas as pl
from jax.experimental.pallas import tpu as pltpu
from jax.experimental.pallas import tpu_sc as plsc

_NC = 2
_NS = 16
_L = 16
_NW = _NC * _NS

_MESH = dict(core_axis_name="c", subcore_axis_name="s")


def _degree_partials(ei5, n_pad):
    _, nw, nwin, wchunk, ch = ei5.shape
    zps = n_pad // _NS

    @functools.partial(
        pl.kernel,
        out_type=jax.ShapeDtypeStruct((_NC * n_pad,), jnp.float32),
        mesh=plsc.VectorSubcoreMesh(**_MESH),
        scratch_types=[
            pltpu.VMEM((wchunk, ch), jnp.int32),
            pltpu.VMEM((wchunk, ch), jnp.int32),
            pltpu.VMEM((ch,), jnp.float32),
            pltpu.VMEM((zps,), jnp.float32),
            pltpu.VMEM_SHARED((n_pad,), jnp.float32),
            pltpu.SemaphoreType.DMA,
            pltpu.SemaphoreType.DMA,
        ],
    )
    def deg_kernel(ei_hbm, out_hbm, cid0, cid1, ones_v, zeros_v, deg_sh,
                   semi, sems):
        cid = lax.axis_index("c")
        sid = lax.axis_index("s")
        wid = cid * _NS + sid
        cbufs = (cid0, cid1)
        pltpu.async_copy(ei_hbm.at[1, wid, 0], cid0, semi)

        @pl.loop(0, ch, step=_L)
        def _(i):
            ones_v[pl.ds(i, _L)] = jnp.ones((_L,), jnp.float32)

        @pl.loop(0, zps, step=_L)
        def _(i):
            zeros_v[pl.ds(i, _L)] = jnp.zeros((_L,), jnp.float32)

        pltpu.sync_copy(zeros_v, deg_sh.at[pl.ds(sid * zps, zps)])
        plsc.subcore_barrier()
        pltpu.make_async_copy(ei_hbm.at[1, wid, 0], cid0, semi).wait()

        for w in range(nwin):
            cbuf = cbufs[w % 2]
            if w + 1 < nwin:
                pltpu.async_copy(ei_hbm.at[1, wid, w + 1],
                                 cbufs[(w + 1) % 2], semi)

            @pl.loop(0, wchunk)
            def _(j):
                pltpu.async_copy(ones_v, deg_sh.at[cbuf.at[j]], sems,
                                 add=True)

            @pl.loop(0, wchunk)
            def _(j):
                pltpu.make_async_copy(ones_v, deg_sh.at[cbuf.at[j]],
                                      sems).wait()

            if w + 1 < nwin:
                pltpu.make_async_copy(ei_hbm.at[1, wid, w + 1],
                                      cbufs[(w + 1) % 2], semi).wait()

        plsc.subcore_barrier()
        pltpu.sync_copy(deg_sh.at[pl.ds(sid * zps, zps)], zeros_v)
        pltpu.sync_copy(zeros_v, out_hbm.at[pl.ds(cid * n_pad + sid * zps, zps)])

    return deg_kernel(ei5)


def _aggregate(y, ei5):
    n_acc, d = y.shape
    _, nw, nwin, wchunk, ch = ei5.shape
    npc = n_acc // _NS
    nring = 4
    rem = wchunk % nring
    main_hi = wchunk - nring - rem

    @functools.partial(
        pl.kernel,
        out_type=jax.ShapeDtypeStruct((_NC, n_acc, d), jnp.float32),
        mesh=plsc.VectorSubcoreMesh(**_MESH),
        scratch_types=[
            pltpu.VMEM((wchunk, ch), jnp.int32),
            pltpu.VMEM((wchunk, ch), jnp.int32),
            pltpu.VMEM((ch, d), jnp.float32),
            pltpu.VMEM((ch, d), jnp.float32),
            pltpu.VMEM((ch, d), jnp.float32),
            pltpu.VMEM((ch, d), jnp.float32),
            pltpu.VMEM_SHARED((n_acc, d), jnp.float32),
            pltpu.SemaphoreType.DMA,
            pltpu.SemaphoreType.DMA,
            pltpu.SemaphoreType.DMA,
            pltpu.SemaphoreType.DMA,
            pltpu.SemaphoreType.DMA,
        ],
    )
    def agg_kernel(y_hbm, ei_hbm, out_hbm,
                   ridx, cidx, b0, b1, b2, b3, agg_sh,
                   semi, s0, s1, s2, s3):
        bufs = (b0, b1, b2, b3)
        sems = (s0, s1, s2, s3)
        cid = lax.axis_index("c")
        sid = lax.axis_index("s")
        wid = cid * _NS + sid
        pltpu.async_copy(ei_hbm.at[0, wid, 0], ridx, semi)
        pltpu.async_copy(ei_hbm.at[1, wid, 0], cidx, semi)
        pltpu.make_async_copy(ei_hbm.at[0, wid, 0], ridx, semi).wait()
        for k in range(1, nring):
            pltpu.async_copy(y_hbm.at[ridx.at[k]], bufs[k], sems[k])

        @pl.loop(0, ch)
        def _(r):
            @pl.loop(0, d, step=_L)
            def _(c0):
                b0[r, pl.ds(c0, _L)] = jnp.zeros((_L,), jnp.float32)

        @pl.loop(0, npc, step=ch)
        def _(r0):
            pltpu.sync_copy(b0, agg_sh.at[pl.ds(sid * npc + r0, ch)])

        pltpu.async_copy(y_hbm.at[ridx.at[0]], b0, sems[0])
        plsc.subcore_barrier()
        pltpu.make_async_copy(ei_hbm.at[1, wid, 0], cidx, semi).wait()

        @pl.loop(0, nwin)
        def _(w):
            @pl.when(w > 0)
            def _():
                for k in range(nring):
                    pltpu.async_copy(y_hbm.at[ridx.at[k]], bufs[k], sems[k])

            @pl.loop(0, main_hi, step=nring)
            def _(j):
                for k in range(nring):
                    pltpu.make_async_copy(y_hbm.at[ridx.at[j + k]], bufs[k],
                                          sems[k]).wait()
                    pltpu.sync_copy(bufs[k], agg_sh.at[cidx.at[j + k]],
                                    add=True)
                    pltpu.async_copy(y_hbm.at[ridx.at[j + k + nring]],
                                     bufs[k], sems[k])

            for k in range(nring):
                pltpu.make_async_copy(y_hbm.at[ridx.at[main_hi + k]],
                                      bufs[k], sems[k]).wait()
                pltpu.sync_copy(bufs[k], agg_sh.at[cidx.at[main_hi + k]],
                                add=True)
                if k < rem:
                    pltpu.async_copy(
                        y_hbm.at[ridx.at[main_hi + nring + k]],
                        bufs[k], sems[k])
            for k in range(rem):
                pltpu.make_async_copy(y_hbm.at[ridx.at[main_hi + nring + k]],
                                      bufs[k], sems[k]).wait()
                pltpu.sync_copy(bufs[k],
                                agg_sh.at[cidx.at[main_hi + nring + k]],
                                add=True)

            @pl.when(w + 1 < nwin)
            def _():
                pltpu.sync_copy(ei_hbm.at[0, wid, w + 1], ridx)
                pltpu.sync_copy(ei_hbm.at[1, wid, w + 1], cidx)

        plsc.subcore_barrier()

        @pl.loop(0, npc, step=4 * ch)
        def _(r0):
            for k in range(4):
                pltpu.async_copy(
                    agg_sh.at[pl.ds(sid * npc + r0 + k * ch, ch)],
                    bufs[k], sems[k])
            for k in range(4):
                pltpu.make_async_copy(
                    agg_sh.at[pl.ds(sid * npc + r0 + k * ch, ch)],
                    bufs[k], sems[k]).wait()
                pltpu.async_copy(
                    bufs[k],
                    out_hbm.at[cid, pl.ds(sid * npc + r0 + k * ch, ch)],
                    sems[k])
            for k in range(4):
                pltpu.make_async_copy(
                    bufs[k],
                    out_hbm.at[cid, pl.ds(sid * npc + r0 + k * ch, ch)],
                    sems[k]).wait()

    return agg_kernel(y, ei5)


_BN = 2048


def _mm_scale_body(x_ref, w_ref, dp_ref, y_ref, dis_ref):
    i = pl.program_id(0)
    n_acc = dp_ref.shape[0] // 2
    deg = (dp_ref[pl.ds(i * _BN, _BN)]
           + dp_ref[pl.ds(n_acc + i * _BN, _BN)] + 1.0)
    dis = lax.rsqrt(deg)[:, None]
    y_ref[...] = jnp.dot(x_ref[...].astype(jnp.bfloat16),
                         w_ref[...].astype(jnp.bfloat16),
                         preferred_element_type=jnp.float32) * dis
    dis_ref[...] = dis


def _mm_scale(x, w, degp, n_acc):
    n, din = x.shape
    dout = w.shape[1]
    return pl.pallas_call(
        _mm_scale_body,
        grid=(n_acc // _BN,),
        in_specs=[pl.BlockSpec((_BN, din), lambda i: (i, 0)),
                  pl.BlockSpec((din, dout), lambda i: (0, 0)),
                  pl.BlockSpec((2 * n_acc,), lambda i: (0,))],
        out_specs=[pl.BlockSpec((_BN, dout), lambda i: (i, 0)),
                   pl.BlockSpec((_BN, 1), lambda i: (i, 0))],
        out_shape=[jax.ShapeDtypeStruct((n_acc, dout), jnp.float32),
                   jax.ShapeDtypeStruct((n_acc, 1), jnp.float32)],
    )(x, w, degp)


def _mid_body(y_ref, p0_ref, p1_ref, dis_ref, b_ref, w_ref, o_ref):
    dis = dis_ref[...]
    h = (y_ref[...] + p0_ref[0] + p1_ref[0]) * dis + b_ref[...]
    o_ref[...] = jnp.dot(h.astype(jnp.bfloat16),
                         w_ref[...].astype(jnp.bfloat16),
                         preferred_element_type=jnp.float32) * dis


def _mid(y, p, dis, b, w):
    n_acc, d = y.shape
    dout = w.shape[1]
    blk2 = pl.BlockSpec((_BN, d), lambda i: (i, 0))
    return pl.pallas_call(
        _mid_body,
        grid=(n_acc // _BN,),
        in_specs=[blk2,
                  pl.BlockSpec((1, _BN, d), lambda i: (0, i, 0)),
                  pl.BlockSpec((1, _BN, d), lambda i: (1, i, 0)),
                  pl.BlockSpec((_BN, 1), lambda i: (i, 0)),
                  pl.BlockSpec((1, d), lambda i: (0, 0)),
                  pl.BlockSpec((d, dout), lambda i: (0, 0))],
        out_specs=pl.BlockSpec((_BN, dout), lambda i: (i, 0)),
        out_shape=jax.ShapeDtypeStruct((n_acc, dout), jnp.float32),
    )(y, p, p, dis, b, w)


_BNF = 2000


def _final_body(y_ref, q0_ref, q1_ref, dis_ref, b_ref, o_ref):
    o_ref[...] = (y_ref[...] + q0_ref[0] + q1_ref[0]) * dis_ref[...] \
        + b_ref[...]


def _final(y, q, dis, b, n):
    n_acc, d = y.shape
    blk2 = pl.BlockSpec((_BNF, d), lambda i: (i, 0))
    return pl.pallas_call(
        _final_body,
        grid=(n // _BNF,),
        in_specs=[blk2,
                  pl.BlockSpec((1, _BNF, d), lambda i: (0, i, 0)),
                  pl.BlockSpec((1, _BNF, d), lambda i: (1, i, 0)),
                  pl.BlockSpec((_BNF, 1), lambda i: (i, 0)),
                  pl.BlockSpec((1, d), lambda i: (0, 0))],
        out_specs=pl.BlockSpec((_BNF, d), lambda i: (i, 0)),
        out_shape=jax.ShapeDtypeStruct((n, d), jnp.float32),
    )(y, q, q, dis, b)


def kernel(x, edge_index, W1, b1, W2, b2):
    n, _ = x.shape
    e = edge_index.shape[1]
    epw = e // _NW
    ch = 80
    nwin = 5
    wchunk = epw // ch // nwin
    ei5 = edge_index.reshape(2, _NW, nwin, wchunk, ch)
    n_acc = -(-n // (_NS * 128)) * (_NS * 128)

    degp = _degree_partials(ei5, n_acc)
    y1, dis = _mm_scale(x, W1, degp, n_acc)
    p = _aggregate(y1, ei5)
    y2 = _mid(y1, p, dis, b1.reshape(1, -1), W2)
    q = _aggregate(y2, ei5)
    return _final(y2, q, dis, b2.reshape(1, -1), n)
